# baseline (speedup 1.0000x reference)
import functools

import jax
import jax.numpy as jnp
from jax import lax
from jax.experimental import pallas as pl
from jax.experimental.pallas import tpu as pltpu
from jax.experimental.pallas import tpu_sc as plsc

NC = 2
NS = 16
NW = NC * NS
CH = 128


@functools.partial(jax.jit, static_argnames=("n_per_w", "n_chunks"))
def _emb_lookup(idx3, W, n_per_w, n_chunks):
    D = W.shape[1]

    mesh = plsc.VectorSubcoreMesh(
        core_axis_name="c", subcore_axis_name="s",
        num_cores=NC, num_subcores=NS,
    )

    @functools.partial(
        pl.kernel,
        out_type=jax.ShapeDtypeStruct((NW * n_per_w, D), jnp.float32),
        mesh=mesh,
        scratch_types=[
            pltpu.VMEM((n_chunks, CH), jnp.int32),
            pltpu.VMEM((CH, D), jnp.float32),
            pltpu.SemaphoreType.DMA,
        ],
        compiler_params=pltpu.CompilerParams(use_tc_tiling_on_sc=False),
    )
    def k(idx_hbm, table_hbm, out_hbm, idx_v, buf, gsem):
        cid = lax.axis_index("c")
        sid = lax.axis_index("s")
        wid = sid * NC + cid
        base = wid * n_per_w

        pltpu.sync_copy(idx_hbm.at[wid], idx_v)

        @pl.loop(0, n_chunks)
        def _(j):
            pltpu.async_copy(table_hbm.at[idx_v.at[j]], buf, gsem).wait()
            pltpu.sync_copy(buf, out_hbm.at[pl.ds(base + j * CH, CH)])

    return k(idx3, W)


def kernel(input, W):
    B, F = input.shape
    D = W.shape[1]
    N = B * F
    assert N % (NW * CH) == 0
    n_per_w = N // NW
    n_chunks = n_per_w // CH
    idx3 = input.reshape(NW, n_chunks, CH).astype(jnp.int32)
    out = _emb_lookup(idx3, W, n_per_w, n_chunks)
    return out.reshape(B, F, D)


# speedup vs baseline: 1.0802x; 1.0802x over previous
import functools

import jax
import jax.numpy as jnp
from jax import lax
from jax.experimental import pallas as pl
from jax.experimental.pallas import tpu as pltpu
from jax.experimental.pallas import tpu_sc as plsc

NC = 2
NS = 16
NW = NC * NS
CH = 128


@functools.partial(jax.jit, static_argnames=("n_per_w", "n_chunks"))
def _emb_lookup(idx3, W, n_per_w, n_chunks):
    D = W.shape[1]

    mesh = plsc.VectorSubcoreMesh(
        core_axis_name="c", subcore_axis_name="s",
        num_cores=NC, num_subcores=NS,
    )

    NBUF = 4
    G = 2
    STEP = G * CH
    n_steps = n_per_w // STEP
    assert n_steps % NBUF == 0 and n_steps >= 2 * NBUF

    @functools.partial(
        pl.kernel,
        out_type=jax.ShapeDtypeStruct((NW * n_per_w, D), jnp.float32),
        mesh=mesh,
        scratch_types=[
            pltpu.VMEM((n_chunks, CH), jnp.int32),
            [pltpu.VMEM((STEP, D), jnp.float32)] * NBUF,
            [pltpu.SemaphoreType.DMA] * NBUF,
            [pltpu.SemaphoreType.DMA] * NBUF,
        ],
        compiler_params=pltpu.CompilerParams(use_tc_tiling_on_sc=False),
    )
    def k(idx_hbm, table_hbm, out_hbm, idx_v, bufs, gsems, ssems):
        cid = lax.axis_index("c")
        sid = lax.axis_index("s")
        wid = sid * NC + cid
        base = wid * n_per_w

        def issue_gather(step, b):
            for g in range(G):
                pltpu.async_copy(
                    table_hbm.at[idx_v.at[step * G + g]],
                    bufs[b].at[pl.ds(g * CH, CH)],
                    gsems[b],
                )

        def wait_gather(b):
            pltpu.make_async_copy(
                table_hbm.at[pl.ds(0, STEP)], bufs[b], gsems[b]
            ).wait()

        def issue_scatter(step, b):
            pltpu.async_copy(
                bufs[b], out_hbm.at[pl.ds(base + step * STEP, STEP)], ssems[b]
            )

        def wait_scatter(b):
            pltpu.make_async_copy(
                bufs[b], out_hbm.at[pl.ds(base, STEP)], ssems[b]
            ).wait()

        pltpu.sync_copy(idx_hbm.at[wid], idx_v)

        for b in range(NBUF):
            issue_gather(b, b)

        @pl.loop(0, n_steps, step=NBUF)
        def _(o):
            for b in range(NBUF):
                s = o + b
                wait_gather(b)
                issue_scatter(s, b)

                @pl.when(s + NBUF < n_steps)
                def _():
                    wait_scatter(b)
                    issue_gather(s + NBUF, b)

        for b in range(NBUF):
            wait_scatter(b)

    return k(idx3, W)


def kernel(input, W):
    B, F = input.shape
    D = W.shape[1]
    N = B * F
    assert N % (NW * CH) == 0
    n_per_w = N // NW
    n_chunks = n_per_w // CH
    idx3 = input.reshape(NW, n_chunks, CH).astype(jnp.int32)
    out = _emb_lookup(idx3, W, n_per_w, n_chunks)
    return out.reshape(B, F, D)
